# Initial kernel scaffold; baseline (speedup 1.0000x reference)
import jax
import jax.numpy as jnp
from jax.experimental import pallas as pl


def kernel(objectness, pred_bbox_deltas, anchors):
    raise NotImplementedError("write your pallas kernel here")


# speedup vs baseline: 18.6368x; 18.6368x over previous
import functools

import jax
import jax.numpy as jnp
from jax import lax
from jax.experimental import pallas as pl
from jax.experimental.pallas import tpu as pltpu

PRE_NMS_TOP_N = 2000
POST_NMS_TOP_N = 1000
NMS_THRESH = 0.7
MIN_SIZE = 1.0
IMG_H = 1024.0
IMG_W = 1024.0
BBOX_XFORM_CLIP = float(jnp.log(1000.0 / 16.0))

T = 2048
BLK = 128
NBLK = T // BLK
OUT_PAD = 1024


def _pair_iou_mask(x1a, y1a, x2a, y2a, ra, x1b, y1b, x2b, y2b, rb):
    ax1 = x1a[:, :, None]
    ay1 = y1a[:, :, None]
    ax2 = x2a[:, :, None]
    ay2 = y2a[:, :, None]
    bx1 = x1b[:, None, :]
    by1 = y1b[:, None, :]
    bx2 = x2b[:, None, :]
    by2 = y2b[:, None, :]
    iw = jnp.maximum(jnp.minimum(ax2, bx2) - jnp.maximum(ax1, bx1), 0.0)
    ih = jnp.maximum(jnp.minimum(ay2, by2) - jnp.maximum(ay1, by1), 0.0)
    inter = iw * ih
    iou = inter / (ra[:, :, None] + rb[:, None, :] - inter + 1e-9)
    return (iou > NMS_THRESH).astype(jnp.float32)


def _nms_kernel(scr_ref, dx_ref, dy_ref, dw_ref, dh_ref,
                a1_ref, a2_ref, a3_ref, a4_ref,
                out_ref, m_ref, keep_ref):
    B = scr_ref.shape[0]

    aw = a3_ref[...] - a1_ref[...]
    ah = a4_ref[...] - a2_ref[...]
    acx = a1_ref[...] + 0.5 * aw
    acy = a2_ref[...] + 0.5 * ah
    dw = jnp.minimum(dw_ref[...], BBOX_XFORM_CLIP)
    dh = jnp.minimum(dh_ref[...], BBOX_XFORM_CLIP)
    pcx = dx_ref[...] * aw + acx
    pcy = dy_ref[...] * ah + acy
    pw = jnp.exp(dw) * aw
    ph = jnp.exp(dh) * ah
    x1 = jnp.clip(pcx - 0.5 * pw, 0.0, IMG_W)
    y1 = jnp.clip(pcy - 0.5 * ph, 0.0, IMG_H)
    x2 = jnp.clip(pcx + 0.5 * pw, 0.0, IMG_W)
    y2 = jnp.clip(pcy + 0.5 * ph, 0.0, IMG_H)
    score = 1.0 / (1.0 + jnp.exp(-scr_ref[...]))
    valid = ((x2 - x1 >= MIN_SIZE) & (y2 - y1 >= MIN_SIZE) & (score > 0.0))
    keep_ref[...] = valid.astype(jnp.float32)
    area = (x2 - x1) * (y2 - y1)

    lane = lax.broadcasted_iota(jnp.int32, (B, BLK), 1)

    for b in range(NBLK):
        sl = slice(b * BLK, (b + 1) * BLK)
        bx1, by1, bx2, by2, br = x1[:, sl], y1[:, sl], x2[:, sl], y2[:, sl], area[:, sl]
        m_ref[...] = _pair_iou_mask(bx1, by1, bx2, by2, br,
                                    bx1, by1, bx2, by2, br)

        def body(j, kb):
            kj = jnp.sum(kb * (lane == j).astype(jnp.float32), axis=1,
                         keepdims=True)
            row = m_ref[:, pl.ds(j, 1), :][:, 0, :]
            sup = row * kj * (lane > j).astype(jnp.float32)
            return kb * (1.0 - sup)

        kb = lax.fori_loop(0, BLK, body, keep_ref[:, sl])
        keep_ref[:, sl] = kb

        for c in range(b + 1, NBLK):
            slc = slice(c * BLK, (c + 1) * BLK)
            m = _pair_iou_mask(bx1, by1, bx2, by2, br,
                               x1[:, slc], y1[:, slc], x2[:, slc], y2[:, slc],
                               area[:, slc])
            hit = jnp.max(m * kb[:, :, None], axis=1)
            keep_ref[:, slc] = keep_ref[:, slc] * (1.0 - hit)

    keep = keep_ref[...]
    csum = keep
    for sh in (1, 2, 4, 8, 16, 32, 64, 128, 256, 512, 1024):
        csum = csum + jnp.concatenate(
            [jnp.zeros((B, sh), jnp.float32), csum[:, :T - sh]], axis=1)
    pos = csum - 1.0

    zeros = jnp.zeros_like(score)
    data = jnp.stack([x1, y1, x2, y2, score, zeros, zeros, zeros], axis=1)
    data = data * keep[:, None, :]

    for o in range(OUT_PAD // BLK):
        tgt = (jnp.float32(o * BLK)
               + lax.broadcasted_iota(jnp.int32, (1, 1, BLK), 2).astype(jnp.float32))
        onehot = (pos[:, :, None] == tgt).astype(jnp.float32) * keep[:, :, None]
        out_ref[:, :, o * BLK:(o + 1) * BLK] = lax.dot_general(
            data, onehot,
            dimension_numbers=(((2,), (1,)), ((0,), (0,))),
            preferred_element_type=jnp.float32)


def _run_nms(scr, dx, dy, dw, dh, a1, a2, a3, a4):
    B = scr.shape[0]
    return pl.pallas_call(
        _nms_kernel,
        out_shape=jax.ShapeDtypeStruct((B, 8, OUT_PAD), jnp.float32),
        scratch_shapes=[
            pltpu.VMEM((B, BLK, BLK), jnp.float32),
            pltpu.VMEM((B, T), jnp.float32),
        ],
    )(scr, dx, dy, dw, dh, a1, a2, a3, a4)


def kernel(objectness, pred_bbox_deltas, anchors):
    B = objectness.shape[0]
    top_scores, top_idx = lax.top_k(objectness, PRE_NMS_TOP_N)
    deltas = jnp.take_along_axis(pred_bbox_deltas, top_idx[..., None], axis=1)
    anc = anchors[top_idx]

    pad = T - PRE_NMS_TOP_N
    scr = jnp.pad(top_scores, ((0, 0), (0, pad)), constant_values=-1e9)
    dpad = jnp.pad(deltas, ((0, 0), (0, pad), (0, 0)))
    apad = jnp.pad(anc, ((0, 0), (0, pad), (0, 0)))

    out = _run_nms(scr,
                   dpad[..., 0], dpad[..., 1], dpad[..., 2], dpad[..., 3],
                   apad[..., 0], apad[..., 1], apad[..., 2], apad[..., 3])
    out = jnp.transpose(out, (0, 2, 1))[:, :POST_NMS_TOP_N, :5]
    return out


# speedup vs baseline: 22.8300x; 1.2250x over previous
import functools
import math

import jax
import jax.numpy as jnp
from jax import lax
from jax.experimental import pallas as pl
from jax.experimental.pallas import tpu as pltpu

PRE_NMS_TOP_N = 2000
POST_NMS_TOP_N = 1000
NMS_THRESH = 0.7
MIN_SIZE = 1.0
IMG_H = 1024.0
IMG_W = 1024.0
BBOX_XFORM_CLIP = float(math.log(1000.0 / 16.0))

T = 2048
BLK = 128
NBLK = T // BLK
OUT_PAD = 1024


def _pair_iou_mask(x1a, y1a, x2a, y2a, ra, x1b, y1b, x2b, y2b, rb):
    ax1 = x1a[:, :, None]
    ay1 = y1a[:, :, None]
    ax2 = x2a[:, :, None]
    ay2 = y2a[:, :, None]
    bx1 = x1b[:, None, :]
    by1 = y1b[:, None, :]
    bx2 = x2b[:, None, :]
    by2 = y2b[:, None, :]
    iw = jnp.maximum(jnp.minimum(ax2, bx2) - jnp.maximum(ax1, bx1), 0.0)
    ih = jnp.maximum(jnp.minimum(ay2, by2) - jnp.maximum(ay1, by1), 0.0)
    inter = iw * ih
    iou = inter / (ra[:, :, None] + rb[:, None, :] - inter + 1e-9)
    return (iou > NMS_THRESH).astype(jnp.float32)


def _nms_kernel(scr_ref, dx_ref, dy_ref, dw_ref, dh_ref,
                a1_ref, a2_ref, a3_ref, a4_ref,
                out_ref, m_ref, keep_ref):
    B = scr_ref.shape[0]

    aw = a3_ref[...] - a1_ref[...]
    ah = a4_ref[...] - a2_ref[...]
    acx = a1_ref[...] + 0.5 * aw
    acy = a2_ref[...] + 0.5 * ah
    dw = jnp.minimum(dw_ref[...], BBOX_XFORM_CLIP)
    dh = jnp.minimum(dh_ref[...], BBOX_XFORM_CLIP)
    pcx = dx_ref[...] * aw + acx
    pcy = dy_ref[...] * ah + acy
    pw = jnp.exp(dw) * aw
    ph = jnp.exp(dh) * ah
    x1 = jnp.clip(pcx - 0.5 * pw, 0.0, IMG_W)
    y1 = jnp.clip(pcy - 0.5 * ph, 0.0, IMG_H)
    x2 = jnp.clip(pcx + 0.5 * pw, 0.0, IMG_W)
    y2 = jnp.clip(pcy + 0.5 * ph, 0.0, IMG_H)
    score = 1.0 / (1.0 + jnp.exp(-scr_ref[...]))
    valid = ((x2 - x1 >= MIN_SIZE) & (y2 - y1 >= MIN_SIZE) & (score > 0.0))
    keep_ref[...] = valid.astype(jnp.float32)
    area = (x2 - x1) * (y2 - y1)

    lane = lax.broadcasted_iota(jnp.int32, (B, BLK), 1)

    done = jnp.zeros((), jnp.bool_)
    for b in range(NBLK):
        sl = slice(b * BLK, (b + 1) * BLK)

        @pl.when(jnp.logical_not(done))
        def _():
            bx1, by1, bx2, by2, br = (x1[:, sl], y1[:, sl], x2[:, sl],
                                      y2[:, sl], area[:, sl])
            m_ref[...] = _pair_iou_mask(bx1, by1, bx2, by2, br,
                                        bx1, by1, bx2, by2, br)

            def body(j, kb):
                kj = jnp.sum(kb * (lane == j).astype(jnp.float32), axis=1,
                             keepdims=True)
                row = m_ref[:, pl.ds(j, 1), :][:, 0, :]
                sup = row * kj * (lane > j).astype(jnp.float32)
                return kb * (1.0 - sup)

            kb = lax.fori_loop(0, BLK, body, keep_ref[:, sl])
            keep_ref[:, sl] = kb

            for c in range(b + 1, NBLK):
                slc = slice(c * BLK, (c + 1) * BLK)
                m = _pair_iou_mask(bx1, by1, bx2, by2, br,
                                   x1[:, slc], y1[:, slc], x2[:, slc],
                                   y2[:, slc], area[:, slc])
                hit = jnp.max(m * kb[:, :, None], axis=1)
                keep_ref[:, slc] = keep_ref[:, slc] * (1.0 - hit)

        if (b + 1) * BLK >= OUT_PAD and b < NBLK - 1:
            cnt = jnp.sum(keep_ref[:, :(b + 1) * BLK], axis=1)
            done = jnp.logical_or(done, jnp.min(cnt) >= OUT_PAD)

    keep = keep_ref[...]
    csum = keep
    for sh in (1, 2, 4, 8, 16, 32, 64, 128, 256, 512, 1024):
        csum = csum + jnp.concatenate(
            [jnp.zeros((B, sh), jnp.float32), csum[:, :T - sh]], axis=1)
    pos = csum - 1.0

    zeros = jnp.zeros_like(score)
    data = jnp.stack([x1, y1, x2, y2, score, zeros, zeros, zeros], axis=1)
    data = data * keep[:, None, :]

    for o in range(OUT_PAD // BLK):
        tgt = (jnp.float32(o * BLK)
               + lax.broadcasted_iota(jnp.int32, (1, 1, BLK), 2).astype(jnp.float32))
        onehot = (pos[:, :, None] == tgt).astype(jnp.float32) * keep[:, :, None]
        out_ref[:, :, o * BLK:(o + 1) * BLK] = lax.dot_general(
            data, onehot,
            dimension_numbers=(((2,), (1,)), ((0,), (0,))),
            preferred_element_type=jnp.float32)


def _run_nms(scr, dx, dy, dw, dh, a1, a2, a3, a4):
    B = scr.shape[0]
    return pl.pallas_call(
        _nms_kernel,
        out_shape=jax.ShapeDtypeStruct((B, 8, OUT_PAD), jnp.float32),
        scratch_shapes=[
            pltpu.VMEM((B, BLK, BLK), jnp.float32),
            pltpu.VMEM((B, T), jnp.float32),
        ],
    )(scr, dx, dy, dw, dh, a1, a2, a3, a4)


def kernel(objectness, pred_bbox_deltas, anchors):
    B = objectness.shape[0]
    top_scores, top_idx = lax.top_k(objectness, PRE_NMS_TOP_N)
    deltas = jnp.take_along_axis(pred_bbox_deltas, top_idx[..., None], axis=1)
    anc = anchors[top_idx]

    pad = T - PRE_NMS_TOP_N
    scr = jnp.pad(top_scores, ((0, 0), (0, pad)), constant_values=-1e9)
    dpad = jnp.pad(deltas, ((0, 0), (0, pad), (0, 0)))
    apad = jnp.pad(anc, ((0, 0), (0, pad), (0, 0)))

    out = _run_nms(scr,
                   dpad[..., 0], dpad[..., 1], dpad[..., 2], dpad[..., 3],
                   apad[..., 0], apad[..., 1], apad[..., 2], apad[..., 3])
    out = jnp.transpose(out, (0, 2, 1))[:, :POST_NMS_TOP_N, :5]
    return out


# speedup vs baseline: 23.0802x; 1.0110x over previous
import functools
import math

import jax
import jax.numpy as jnp
from jax import lax
from jax.experimental import pallas as pl
from jax.experimental.pallas import tpu as pltpu

PRE_NMS_TOP_N = 2000
POST_NMS_TOP_N = 1000
NMS_THRESH = 0.7
MIN_SIZE = 1.0
IMG_H = 1024.0
IMG_W = 1024.0
BBOX_XFORM_CLIP = float(math.log(1000.0 / 16.0))

T = 2048
BLK = 128
NBLK = T // BLK
OUT_PAD = 1024


def _pair_iou_mask(x1a, y1a, x2a, y2a, ra, x1b, y1b, x2b, y2b, rb):
    ax1 = x1a[:, :, None]
    ay1 = y1a[:, :, None]
    ax2 = x2a[:, :, None]
    ay2 = y2a[:, :, None]
    bx1 = x1b[:, None, :]
    by1 = y1b[:, None, :]
    bx2 = x2b[:, None, :]
    by2 = y2b[:, None, :]
    iw = jnp.maximum(jnp.minimum(ax2, bx2) - jnp.maximum(ax1, bx1), 0.0)
    ih = jnp.maximum(jnp.minimum(ay2, by2) - jnp.maximum(ay1, by1), 0.0)
    inter = iw * ih
    iou = inter / (ra[:, :, None] + rb[:, None, :] - inter + 1e-9)
    return (iou > NMS_THRESH).astype(jnp.float32)


def _nms_kernel(scr_ref, d_ref, a_ref, out_ref, m_ref, keep_ref):
    B = scr_ref.shape[0]
    npad = T - scr_ref.shape[1]

    def _pad(v, const=0.0):
        return jnp.concatenate(
            [v, jnp.full((B, npad), const, jnp.float32)], axis=1)

    scr = _pad(scr_ref[...], -1e9)
    a1 = _pad(a_ref[:, 0, :])
    a2 = _pad(a_ref[:, 1, :])
    a3 = _pad(a_ref[:, 2, :])
    a4 = _pad(a_ref[:, 3, :])

    aw = a3 - a1
    ah = a4 - a2
    acx = a1 + 0.5 * aw
    acy = a2 + 0.5 * ah
    dw = jnp.minimum(_pad(d_ref[:, 2, :]), BBOX_XFORM_CLIP)
    dh = jnp.minimum(_pad(d_ref[:, 3, :]), BBOX_XFORM_CLIP)
    pcx = _pad(d_ref[:, 0, :]) * aw + acx
    pcy = _pad(d_ref[:, 1, :]) * ah + acy
    pw = jnp.exp(dw) * aw
    ph = jnp.exp(dh) * ah
    x1 = jnp.clip(pcx - 0.5 * pw, 0.0, IMG_W)
    y1 = jnp.clip(pcy - 0.5 * ph, 0.0, IMG_H)
    x2 = jnp.clip(pcx + 0.5 * pw, 0.0, IMG_W)
    y2 = jnp.clip(pcy + 0.5 * ph, 0.0, IMG_H)
    score = 1.0 / (1.0 + jnp.exp(-scr))
    valid = ((x2 - x1 >= MIN_SIZE) & (y2 - y1 >= MIN_SIZE) & (score > 0.0))
    keep_ref[...] = valid.astype(jnp.float32)
    area = (x2 - x1) * (y2 - y1)

    lane = lax.broadcasted_iota(jnp.int32, (B, BLK), 1)

    done = jnp.zeros((), jnp.bool_)
    for b in range(NBLK):
        sl = slice(b * BLK, (b + 1) * BLK)

        @pl.when(jnp.logical_not(done))
        def _():
            bx1, by1, bx2, by2, br = (x1[:, sl], y1[:, sl], x2[:, sl],
                                      y2[:, sl], area[:, sl])
            m_ref[...] = _pair_iou_mask(bx1, by1, bx2, by2, br,
                                        bx1, by1, bx2, by2, br)

            def body(j, kb):
                kj = jnp.sum(kb * (lane == j).astype(jnp.float32), axis=1,
                             keepdims=True)
                row = m_ref[:, pl.ds(j, 1), :][:, 0, :]
                sup = row * kj * (lane > j).astype(jnp.float32)
                return kb * (1.0 - sup)

            kb = lax.fori_loop(0, BLK, body, keep_ref[:, sl])
            keep_ref[:, sl] = kb

            for c in range(b + 1, NBLK):
                slc = slice(c * BLK, (c + 1) * BLK)
                m = _pair_iou_mask(bx1, by1, bx2, by2, br,
                                   x1[:, slc], y1[:, slc], x2[:, slc],
                                   y2[:, slc], area[:, slc])
                hit = jnp.max(m * kb[:, :, None], axis=1)
                keep_ref[:, slc] = keep_ref[:, slc] * (1.0 - hit)

        if (b + 1) * BLK >= OUT_PAD and b < NBLK - 1:
            cnt = jnp.sum(keep_ref[:, :(b + 1) * BLK], axis=1)
            done = jnp.logical_or(done, jnp.min(cnt) >= OUT_PAD)

    keep = keep_ref[...]
    csum = keep
    for sh in (1, 2, 4, 8, 16, 32, 64, 128, 256, 512, 1024):
        csum = csum + jnp.concatenate(
            [jnp.zeros((B, sh), jnp.float32), csum[:, :T - sh]], axis=1)
    pos = csum - 1.0

    zeros = jnp.zeros_like(score)
    data = jnp.stack([x1, y1, x2, y2, score, zeros, zeros, zeros], axis=1)
    data = data * keep[:, None, :]

    for o in range(OUT_PAD // BLK):
        tgt = (jnp.float32(o * BLK)
               + lax.broadcasted_iota(jnp.int32, (1, 1, BLK), 2).astype(jnp.float32))
        onehot = (pos[:, :, None] == tgt).astype(jnp.float32) * keep[:, :, None]
        out_ref[:, :, o * BLK:(o + 1) * BLK] = lax.dot_general(
            data, onehot,
            dimension_numbers=(((2,), (1,)), ((0,), (0,))),
            preferred_element_type=jnp.float32)


def _run_nms(scr, deltas_t, anc_t):
    B = scr.shape[0]
    return pl.pallas_call(
        _nms_kernel,
        out_shape=jax.ShapeDtypeStruct((B, 8, OUT_PAD), jnp.float32),
        scratch_shapes=[
            pltpu.VMEM((B, BLK, BLK), jnp.float32),
            pltpu.VMEM((B, T), jnp.float32),
        ],
    )(scr, deltas_t, anc_t)


def kernel(objectness, pred_bbox_deltas, anchors):
    top_scores, top_idx = lax.top_k(objectness, PRE_NMS_TOP_N)
    deltas = jnp.take_along_axis(pred_bbox_deltas, top_idx[..., None], axis=1)
    anc = anchors[top_idx]

    out = _run_nms(top_scores,
                   jnp.transpose(deltas, (0, 2, 1)),
                   jnp.transpose(anc, (0, 2, 1)))
    out = jnp.transpose(out, (0, 2, 1))[:, :POST_NMS_TOP_N, :5]
    return out
